# baseline (speedup 1.0000x reference)
import functools

import jax
import jax.numpy as jnp
from jax import lax
from jax.experimental import pallas as pl
from jax.experimental.pallas import tpu as pltpu
from jax.experimental.pallas import tpu_sc as plsc

N = 10000
D_IN = 128
NHID = 16
HEADS = 8
D = NHID * HEADS

NPAD = 10240
ROWS_PER_TILE = NPAD // 16

E = 320000
NE = E + N
NW = 32
B = 64
NBLK = 162
NE_PAD = NW * NBLK * B
NBLK_ALLOC = NBLK + 2

_f32 = jnp.float32


_BLKA = 1280


def _prep_body(x_ref, w_ref, asrc_ref, adst_ref, h_ref, s16_ref, d16_ref):
    h = jnp.dot(x_ref[...], w_ref[...], preferred_element_type=_f32)
    h_ref[...] = h
    row = lax.broadcasted_iota(jnp.int32, (D, 16), 0)
    col = lax.broadcasted_iota(jnp.int32, (D, 16), 1)
    sel = jnp.where((row // NHID) == col, 1.0, 0.0).astype(_f32)
    s16_ref[...] = jnp.dot(h * asrc_ref[...], sel, preferred_element_type=_f32)
    d16_ref[...] = jnp.dot(h * adst_ref[...], sel, preferred_element_type=_f32)


_prep_call = pl.pallas_call(
    _prep_body,
    grid=(NPAD // _BLKA,),
    in_specs=[
        pl.BlockSpec((_BLKA, D_IN), lambda i: (i, 0)),
        pl.BlockSpec((D_IN, D), lambda i: (0, 0)),
        pl.BlockSpec((1, D), lambda i: (0, 0)),
        pl.BlockSpec((1, D), lambda i: (0, 0)),
    ],
    out_specs=[
        pl.BlockSpec((_BLKA, D), lambda i: (i, 0)),
        pl.BlockSpec((_BLKA, 16), lambda i: (i, 0)),
        pl.BlockSpec((_BLKA, 16), lambda i: (i, 0)),
    ],
    out_shape=[
        jax.ShapeDtypeStruct((NPAD, D), _f32),
        jax.ShapeDtypeStruct((NPAD, 16), _f32),
        jax.ShapeDtypeStruct((NPAD, 16), _f32),
    ],
)


_sc_mesh = plsc.VectorSubcoreMesh(
    core_axis_name="c", subcore_axis_name="s", num_cores=2, num_subcores=16
)


@functools.partial(
    pl.kernel,
    out_type=[
        jax.ShapeDtypeStruct((2, NPAD, D), _f32),
        jax.ShapeDtypeStruct((2, NPAD, 16), _f32),
    ],
    mesh=_sc_mesh,
    compiler_params=pltpu.CompilerParams(use_tc_tiling_on_sc=False),
    scratch_types=(
        [pltpu.VMEM((B,), jnp.int32)] * 6
        + [pltpu.VMEM((B,), jnp.int32)] * 6
        + [pltpu.VMEM((B, 16), _f32)] * 3
        + [pltpu.VMEM((B, 16), _f32)] * 3
        + [pltpu.VMEM((B, 16), _f32)] * 3
        + [pltpu.VMEM((B, D), _f32)] * 3
        + [
            pltpu.VMEM_SHARED((NPAD, D), _f32),
            pltpu.VMEM_SHARED((NPAD, 16), _f32),
        ]
        + [pltpu.SemaphoreType.DMA] * 12
    ),
)
def _edge_call(src_hbm, dst_hbm, h_hbm, s16_hbm, d16_hbm, acc_out, den_out,
               *scr):
    sidx = scr[0:6]
    didx = scr[6:12]
    asv = scr[12:15]
    adv = scr[15:18]
    wv = scr[18:21]
    hr = scr[21:24]
    acc_sh, den_sh = scr[24], scr[25]
    semg = scr[26:29]
    sems = scr[29:32]
    semi = scr[32:38]

    cid = lax.axis_index("c")
    sid = lax.axis_index("s")
    wid = cid * 16 + sid
    base = sid * ROWS_PER_TILE

    def fire_idx(q, a):
        pltpu.async_copy(src_hbm.at[wid, a], sidx[q], semi[q])
        pltpu.async_copy(dst_hbm.at[wid, a], didx[q], semi[q])

    def drain_idx(q):
        pltpu.make_async_copy(src_hbm.at[wid, 0], sidx[q], semi[q]).wait()
        pltpu.make_async_copy(src_hbm.at[wid, 0], didx[q], semi[q]).wait()

    def fire_gathers(b, q):
        pltpu.async_copy(s16_hbm.at[sidx[q]], asv[b], semg[b])
        pltpu.async_copy(d16_hbm.at[didx[q]], adv[b], semg[b])
        pltpu.async_copy(h_hbm.at[sidx[q]], hr[b], semg[b])

    def drain_gathers(b):
        pltpu.make_async_copy(s16_hbm.at[pl.ds(0, B)], asv[b], semg[b]).wait()
        pltpu.make_async_copy(s16_hbm.at[pl.ds(0, B)], adv[b], semg[b]).wait()
        pltpu.make_async_copy(h_hbm.at[pl.ds(0, B)], hr[b], semg[b]).wait()

    def fire_scatters(b, q):
        pltpu.async_copy(wv[b], den_sh.at[didx[q]], sems[b], add=True)
        pltpu.async_copy(hr[b], acc_sh.at[didx[q]], sems[b], add=True)

    def drain_scatters(b):
        pltpu.make_async_copy(s16_hbm.at[pl.ds(0, B)], wv[b], sems[b]).wait()
        pltpu.make_async_copy(h_hbm.at[pl.ds(0, B)], hr[b], sems[b]).wait()

    def compute(b):
        def _cbody(i, carry):
            e = asv[b][i, :] + adv[b][i, :]
            e = jnp.where(e >= 0.0, e, 0.2 * e)
            w16 = jnp.exp(e)
            wv[b][i, :] = w16
            for hh in range(HEADS):
                hr[b][i, pl.ds(hh * NHID, NHID)] = (
                    hr[b][i, pl.ds(hh * NHID, NHID)] * w16[hh]
                )
            return carry

        lax.fori_loop(0, B, _cbody, 0, unroll=2)

    def _zero(i, carry):
        for bb in range(3):
            for c in range(D // 16):
                hr[bb][i, pl.ds(c * 16, 16)] = jnp.zeros((16,), _f32)
            wv[bb][i, :] = jnp.zeros((16,), _f32)
        return carry

    lax.fori_loop(0, B, _zero, 0)
    for r in range(ROWS_PER_TILE // B):
        pltpu.sync_copy(hr[0], acc_sh.at[pl.ds(base + r * B, B)])
        pltpu.sync_copy(wv[0], den_sh.at[pl.ds(base + r * B, B)])
    plsc.subcore_barrier()

    fire_idx(0, 0)
    fire_idx(1, 1)
    drain_idx(0)
    fire_scatters(1, 0)
    fire_scatters(2, 0)
    fire_gathers(0, 0)

    def outer(t, carry):
        a0 = 6 * t
        for k in range(6):
            b = k % 3
            drain_gathers(b)
            drain_scatters((b + 1) % 3)
            drain_idx((k + 1) % 6)
            fire_gathers((b + 1) % 3, (k + 1) % 6)
            compute(b)
            fire_scatters(b, k)
            fire_idx((k + 2) % 6, a0 + k + 2)
        return carry

    lax.fori_loop(0, NBLK // 6, outer, 0)
    drain_scatters(1)
    drain_scatters(2)
    drain_gathers(0)
    drain_idx(1)
    plsc.subcore_barrier()

    pltpu.sync_copy(
        acc_sh.at[pl.ds(base, ROWS_PER_TILE)],
        acc_out.at[cid, pl.ds(base, ROWS_PER_TILE)],
    )
    pltpu.sync_copy(
        den_sh.at[pl.ds(base, ROWS_PER_TILE)],
        den_out.at[cid, pl.ds(base, ROWS_PER_TILE)],
    )


_BLKB = 1000


def _b1_body(acc0_ref, acc1_ref, den0_ref, den1_ref, b_ref, wp_ref,
             wqt_ref, wkt_ref, hp_ref, q_ref, k_ref):
    i = pl.program_id(0)
    acc = acc0_ref[0] + acc1_ref[0]
    den16 = den0_ref[0] + den1_ref[0]
    col = lax.broadcasted_iota(jnp.int32, (16, D), 0)
    row = lax.broadcasted_iota(jnp.int32, (16, D), 1)
    selT = jnp.where((row // NHID) == col, 1.0, 0.0).astype(_f32)
    dexp = jnp.dot(den16, selT, preferred_element_type=_f32)
    gat = acc / (dexp + 1e-16) + b_ref[...]
    hp = lax.dot_general(gat, wp_ref[...], (((1,), (1,)), ((), ())),
                         preferred_element_type=_f32)
    hp_ref[...] = hp
    q = lax.dot_general(hp, wqt_ref[...], (((0,), (0,)), ((), ())),
                        preferred_element_type=_f32)
    k = lax.dot_general(hp, wkt_ref[...], (((0,), (0,)), ((), ())),
                        preferred_element_type=_f32)

    @pl.when(i == 0)
    def _init():
        q_ref[...] = jnp.zeros((D, D), _f32)
        k_ref[...] = jnp.zeros((D, D), _f32)

    q_ref[...] += q
    k_ref[...] += k


_b1_call = pl.pallas_call(
    _b1_body,
    grid=(N // _BLKB,),
    in_specs=[
        pl.BlockSpec((1, _BLKB, D), lambda i: (0, i, 0)),
        pl.BlockSpec((1, _BLKB, D), lambda i: (1, i, 0)),
        pl.BlockSpec((1, _BLKB, 16), lambda i: (0, i, 0)),
        pl.BlockSpec((1, _BLKB, 16), lambda i: (1, i, 0)),
        pl.BlockSpec((1, D), lambda i: (0, 0)),
        pl.BlockSpec((D, D), lambda i: (0, 0)),
        pl.BlockSpec((_BLKB, D), lambda i: (i, 0)),
        pl.BlockSpec((_BLKB, D), lambda i: (i, 0)),
    ],
    out_specs=[
        pl.BlockSpec((_BLKB, D), lambda i: (i, 0)),
        pl.BlockSpec((D, D), lambda i: (0, 0)),
        pl.BlockSpec((D, D), lambda i: (0, 0)),
    ],
    out_shape=[
        jax.ShapeDtypeStruct((N, D), _f32),
        jax.ShapeDtypeStruct((D, D), _f32),
        jax.ShapeDtypeStruct((D, D), _f32),
    ],
)


def _b2_body(q_ref, k_ref, bq_ref, bk_ref, a_ref):
    q_all = q_ref[...] + bq_ref[...]
    k_all = k_ref[...] + bk_ref[...]
    acc = jnp.zeros((D, D), _f32)
    for hh in range(HEADS):
        qh = q_all[:, hh * NHID:(hh + 1) * NHID]
        kh = k_all[:, hh * NHID:(hh + 1) * NHID]
        s = lax.dot_general(qh, kh, (((1,), (1,)), ((), ())),
                            preferred_element_type=_f32) * 0.25
        s = s - jnp.max(s, axis=-1, keepdims=True)
        ex = jnp.exp(s)
        acc = acc + ex / jnp.sum(ex, axis=-1, keepdims=True)
    a_ref[...] = acc


_b2_call = pl.pallas_call(
    _b2_body,
    in_specs=[
        pl.BlockSpec((D, D), lambda: (0, 0)),
        pl.BlockSpec((D, D), lambda: (0, 0)),
        pl.BlockSpec((1, D), lambda: (0, 0)),
        pl.BlockSpec((1, D), lambda: (0, 0)),
    ],
    out_specs=pl.BlockSpec((D, D), lambda: (0, 0)),
    out_shape=jax.ShapeDtypeStruct((D, D), _f32),
)


def _b3_body(hp_ref, a_ref, g_ref, b_ref, o_ref):
    hp = hp_ref[...]
    t = lax.dot_general(hp, a_ref[...], (((1,), (1,)), ((), ())),
                        preferred_element_type=_f32)
    hh = t + hp
    mu = jnp.mean(hh, axis=-1, keepdims=True)
    xc = hh - mu
    var = jnp.mean(xc * xc, axis=-1, keepdims=True)
    o_ref[...] = xc / jnp.sqrt(var + 1e-5) * g_ref[...] + b_ref[...]


_b3_call = pl.pallas_call(
    _b3_body,
    grid=(N // _BLKB,),
    in_specs=[
        pl.BlockSpec((_BLKB, D), lambda i: (i, 0)),
        pl.BlockSpec((D, D), lambda i: (0, 0)),
        pl.BlockSpec((1, D), lambda i: (0, 0)),
        pl.BlockSpec((1, D), lambda i: (0, 0)),
    ],
    out_specs=pl.BlockSpec((_BLKB, D), lambda i: (i, 0)),
    out_shape=jax.ShapeDtypeStruct((N, D), _f32),
)


def kernel(x, edge_index, W, att_src, att_dst, b_gat, Wp, Wq, Wk, bq, bk,
           gamma, beta):
    xpad = jnp.pad(x, ((0, NPAD - N), (0, 0)))
    src = edge_index[0].astype(jnp.int32)
    dst = edge_index[1].astype(jnp.int32)
    loop = jnp.arange(N, dtype=jnp.int32)
    pad = jnp.full((NE_PAD - NE,), N, jnp.int32)
    dummy = jnp.full((NW, NBLK_ALLOC - NBLK, B), N, jnp.int32)
    src3 = jnp.concatenate(
        [jnp.concatenate([src, loop, pad]).reshape(NW, NBLK, B), dummy], axis=1)
    dst3 = jnp.concatenate(
        [jnp.concatenate([dst, loop, pad]).reshape(NW, NBLK, B), dummy], axis=1)

    h, s16, d16 = _prep_call(xpad, W, att_src.reshape(1, D),
                             att_dst.reshape(1, D))
    acc2, den2 = _edge_call(src3, dst3, h, s16, d16)

    wqt = Wq.transpose(1, 0, 2).reshape(N, D)
    wkt = Wk.transpose(1, 0, 2).reshape(N, D)
    hp, q_acc, k_acc = _b1_call(acc2, acc2, den2, den2, b_gat.reshape(1, D),
                                Wp, wqt, wkt)
    a_sum = _b2_call(q_acc, k_acc, bq.reshape(1, D), bk.reshape(1, D))
    return _b3_call(hp, a_sum, gamma.reshape(1, D), beta.reshape(1, D))


# speedup vs baseline: 1.2308x; 1.2308x over previous
import functools

import jax
import jax.numpy as jnp
from jax import lax
from jax.experimental import pallas as pl
from jax.experimental.pallas import tpu as pltpu
from jax.experimental.pallas import tpu_sc as plsc

N = 10000
D_IN = 128
NHID = 16
HEADS = 8
D = NHID * HEADS

NPAD = 10240
ROWS_PER_TILE = NPAD // 16

E = 320000
NE = E + N
NW = 32
B = 64
NBLK = 162
NE_PAD = NW * NBLK * B
NBLK_ALLOC = NBLK + 2

_f32 = jnp.float32


_BLKA = 1280


def _prep_body(x_ref, w_ref, asrc_ref, adst_ref, h_ref, s16_ref, d16_ref):
    h = jnp.dot(x_ref[...], w_ref[...], preferred_element_type=_f32)
    h_ref[...] = h
    row = lax.broadcasted_iota(jnp.int32, (D, 16), 0)
    col = lax.broadcasted_iota(jnp.int32, (D, 16), 1)
    sel = jnp.where((row // NHID) == col, 1.0, 0.0).astype(_f32)
    s16_ref[...] = jnp.dot(h * asrc_ref[...], sel, preferred_element_type=_f32)
    d16_ref[...] = jnp.dot(h * adst_ref[...], sel, preferred_element_type=_f32)


_prep_call = pl.pallas_call(
    _prep_body,
    grid=(NPAD // _BLKA,),
    in_specs=[
        pl.BlockSpec((_BLKA, D_IN), lambda i: (i, 0)),
        pl.BlockSpec((D_IN, D), lambda i: (0, 0)),
        pl.BlockSpec((1, D), lambda i: (0, 0)),
        pl.BlockSpec((1, D), lambda i: (0, 0)),
    ],
    out_specs=[
        pl.BlockSpec((_BLKA, D), lambda i: (i, 0)),
        pl.BlockSpec((_BLKA, 16), lambda i: (i, 0)),
        pl.BlockSpec((_BLKA, 16), lambda i: (i, 0)),
    ],
    out_shape=[
        jax.ShapeDtypeStruct((NPAD, D), _f32),
        jax.ShapeDtypeStruct((NPAD, 16), _f32),
        jax.ShapeDtypeStruct((NPAD, 16), _f32),
    ],
)


_sc_mesh = plsc.VectorSubcoreMesh(
    core_axis_name="c", subcore_axis_name="s", num_cores=2, num_subcores=16
)


@functools.partial(
    pl.kernel,
    out_type=[
        jax.ShapeDtypeStruct((2, NPAD, D), _f32),
        jax.ShapeDtypeStruct((2, NPAD, 16), _f32),
    ],
    mesh=_sc_mesh,
    compiler_params=pltpu.CompilerParams(use_tc_tiling_on_sc=False),
    scratch_types=(
        [pltpu.VMEM((B,), jnp.int32)] * 6
        + [pltpu.VMEM((B,), jnp.int32)] * 6
        + [pltpu.VMEM((B, 16), _f32)] * 3
        + [pltpu.VMEM((B, 16), _f32)] * 3
        + [pltpu.VMEM((B, 16), _f32)] * 3
        + [pltpu.VMEM((B, D), _f32)] * 3
        + [
            pltpu.VMEM_SHARED((NPAD, D), _f32),
            pltpu.VMEM_SHARED((NPAD, 16), _f32),
        ]
        + [pltpu.SemaphoreType.DMA] * 12
    ),
)
def _edge_call(src_hbm, dst_hbm, h_hbm, s16_hbm, d16_hbm, acc_out, den_out,
               *scr):
    sidx = scr[0:6]
    didx = scr[6:12]
    asv = scr[12:15]
    adv = scr[15:18]
    wv = scr[18:21]
    hr = scr[21:24]
    acc_sh, den_sh = scr[24], scr[25]
    semg = scr[26:29]
    sems = scr[29:32]
    semi = scr[32:38]

    cid = lax.axis_index("c")
    sid = lax.axis_index("s")
    wid = cid * 16 + sid
    base = sid * ROWS_PER_TILE

    def fire_idx(q, a):
        pltpu.async_copy(src_hbm.at[wid, a], sidx[q], semi[q])
        pltpu.async_copy(dst_hbm.at[wid, a], didx[q], semi[q])

    def drain_idx(q):
        pltpu.make_async_copy(src_hbm.at[wid, 0], sidx[q], semi[q]).wait()
        pltpu.make_async_copy(src_hbm.at[wid, 0], didx[q], semi[q]).wait()

    def fire_gathers(b, q):
        pltpu.async_copy(s16_hbm.at[sidx[q]], asv[b], semg[b])
        pltpu.async_copy(d16_hbm.at[didx[q]], adv[b], semg[b])
        pltpu.async_copy(h_hbm.at[sidx[q]], hr[b], semg[b])

    def drain_gathers(b):
        pltpu.make_async_copy(s16_hbm.at[pl.ds(0, B)], asv[b], semg[b]).wait()
        pltpu.make_async_copy(s16_hbm.at[pl.ds(0, B)], adv[b], semg[b]).wait()
        pltpu.make_async_copy(h_hbm.at[pl.ds(0, B)], hr[b], semg[b]).wait()

    def fire_scatters(b, q):
        pltpu.async_copy(wv[b], den_sh.at[didx[q]], sems[b], add=True)
        pltpu.async_copy(hr[b], acc_sh.at[didx[q]], sems[b], add=True)

    def drain_scatters(b):
        pltpu.make_async_copy(s16_hbm.at[pl.ds(0, B)], wv[b], sems[b]).wait()
        pltpu.make_async_copy(h_hbm.at[pl.ds(0, B)], hr[b], sems[b]).wait()

    def compute(b):
        @plsc.parallel_loop(0, B, unroll=4)
        def _cbody(i):
            e = asv[b][i, :] + adv[b][i, :]
            e = jnp.where(e >= 0.0, e, 0.2 * e)
            w16 = jnp.exp(e)
            wv[b][i, :] = w16
            for hh in range(HEADS):
                hr[b][i, pl.ds(hh * NHID, NHID)] = (
                    hr[b][i, pl.ds(hh * NHID, NHID)] * w16[hh]
                )

    def _zero(i, carry):
        for bb in range(3):
            for c in range(D // 16):
                hr[bb][i, pl.ds(c * 16, 16)] = jnp.zeros((16,), _f32)
            wv[bb][i, :] = jnp.zeros((16,), _f32)
        return carry

    lax.fori_loop(0, B, _zero, 0)
    for r in range(ROWS_PER_TILE // B):
        pltpu.sync_copy(hr[0], acc_sh.at[pl.ds(base + r * B, B)])
        pltpu.sync_copy(wv[0], den_sh.at[pl.ds(base + r * B, B)])
    plsc.subcore_barrier()

    fire_idx(0, 0)
    fire_idx(1, 1)
    drain_idx(0)
    fire_scatters(1, 0)
    fire_scatters(2, 0)
    fire_gathers(0, 0)

    def outer(t, carry):
        a0 = 6 * t
        for k in range(6):
            b = k % 3
            drain_gathers(b)
            drain_scatters((b + 1) % 3)
            drain_idx((k + 1) % 6)
            fire_gathers((b + 1) % 3, (k + 1) % 6)
            compute(b)
            fire_scatters(b, k)
            fire_idx((k + 2) % 6, a0 + k + 2)
        return carry

    lax.fori_loop(0, NBLK // 6, outer, 0)
    drain_scatters(1)
    drain_scatters(2)
    drain_gathers(0)
    drain_idx(1)
    plsc.subcore_barrier()

    pltpu.sync_copy(
        acc_sh.at[pl.ds(base, ROWS_PER_TILE)],
        acc_out.at[cid, pl.ds(base, ROWS_PER_TILE)],
    )
    pltpu.sync_copy(
        den_sh.at[pl.ds(base, ROWS_PER_TILE)],
        den_out.at[cid, pl.ds(base, ROWS_PER_TILE)],
    )


_BLKB = 1000


def _b1_body(acc0_ref, acc1_ref, den0_ref, den1_ref, b_ref, wp_ref,
             wqt_ref, wkt_ref, hp_ref, q_ref, k_ref):
    i = pl.program_id(0)
    acc = acc0_ref[0] + acc1_ref[0]
    den16 = den0_ref[0] + den1_ref[0]
    col = lax.broadcasted_iota(jnp.int32, (16, D), 0)
    row = lax.broadcasted_iota(jnp.int32, (16, D), 1)
    selT = jnp.where((row // NHID) == col, 1.0, 0.0).astype(_f32)
    dexp = jnp.dot(den16, selT, preferred_element_type=_f32)
    gat = acc / (dexp + 1e-16) + b_ref[...]
    hp = lax.dot_general(gat, wp_ref[...], (((1,), (1,)), ((), ())),
                         preferred_element_type=_f32)
    hp_ref[...] = hp
    q = lax.dot_general(hp, wqt_ref[...], (((0,), (0,)), ((), ())),
                        preferred_element_type=_f32)
    k = lax.dot_general(hp, wkt_ref[...], (((0,), (0,)), ((), ())),
                        preferred_element_type=_f32)

    @pl.when(i == 0)
    def _init():
        q_ref[...] = jnp.zeros((D, D), _f32)
        k_ref[...] = jnp.zeros((D, D), _f32)

    q_ref[...] += q
    k_ref[...] += k


_b1_call = pl.pallas_call(
    _b1_body,
    grid=(N // _BLKB,),
    in_specs=[
        pl.BlockSpec((1, _BLKB, D), lambda i: (0, i, 0)),
        pl.BlockSpec((1, _BLKB, D), lambda i: (1, i, 0)),
        pl.BlockSpec((1, _BLKB, 16), lambda i: (0, i, 0)),
        pl.BlockSpec((1, _BLKB, 16), lambda i: (1, i, 0)),
        pl.BlockSpec((1, D), lambda i: (0, 0)),
        pl.BlockSpec((D, D), lambda i: (0, 0)),
        pl.BlockSpec((_BLKB, D), lambda i: (i, 0)),
        pl.BlockSpec((_BLKB, D), lambda i: (i, 0)),
    ],
    out_specs=[
        pl.BlockSpec((_BLKB, D), lambda i: (i, 0)),
        pl.BlockSpec((D, D), lambda i: (0, 0)),
        pl.BlockSpec((D, D), lambda i: (0, 0)),
    ],
    out_shape=[
        jax.ShapeDtypeStruct((N, D), _f32),
        jax.ShapeDtypeStruct((D, D), _f32),
        jax.ShapeDtypeStruct((D, D), _f32),
    ],
)


def _b2_body(q_ref, k_ref, bq_ref, bk_ref, a_ref):
    q_all = q_ref[...] + bq_ref[...]
    k_all = k_ref[...] + bk_ref[...]
    acc = jnp.zeros((D, D), _f32)
    for hh in range(HEADS):
        qh = q_all[:, hh * NHID:(hh + 1) * NHID]
        kh = k_all[:, hh * NHID:(hh + 1) * NHID]
        s = lax.dot_general(qh, kh, (((1,), (1,)), ((), ())),
                            preferred_element_type=_f32) * 0.25
        s = s - jnp.max(s, axis=-1, keepdims=True)
        ex = jnp.exp(s)
        acc = acc + ex / jnp.sum(ex, axis=-1, keepdims=True)
    a_ref[...] = acc


_b2_call = pl.pallas_call(
    _b2_body,
    in_specs=[
        pl.BlockSpec((D, D), lambda: (0, 0)),
        pl.BlockSpec((D, D), lambda: (0, 0)),
        pl.BlockSpec((1, D), lambda: (0, 0)),
        pl.BlockSpec((1, D), lambda: (0, 0)),
    ],
    out_specs=pl.BlockSpec((D, D), lambda: (0, 0)),
    out_shape=jax.ShapeDtypeStruct((D, D), _f32),
)


def _b3_body(hp_ref, a_ref, g_ref, b_ref, o_ref):
    hp = hp_ref[...]
    t = lax.dot_general(hp, a_ref[...], (((1,), (1,)), ((), ())),
                        preferred_element_type=_f32)
    hh = t + hp
    mu = jnp.mean(hh, axis=-1, keepdims=True)
    xc = hh - mu
    var = jnp.mean(xc * xc, axis=-1, keepdims=True)
    o_ref[...] = xc / jnp.sqrt(var + 1e-5) * g_ref[...] + b_ref[...]


_b3_call = pl.pallas_call(
    _b3_body,
    grid=(N // _BLKB,),
    in_specs=[
        pl.BlockSpec((_BLKB, D), lambda i: (i, 0)),
        pl.BlockSpec((D, D), lambda i: (0, 0)),
        pl.BlockSpec((1, D), lambda i: (0, 0)),
        pl.BlockSpec((1, D), lambda i: (0, 0)),
    ],
    out_specs=pl.BlockSpec((_BLKB, D), lambda i: (i, 0)),
    out_shape=jax.ShapeDtypeStruct((N, D), _f32),
)


def kernel(x, edge_index, W, att_src, att_dst, b_gat, Wp, Wq, Wk, bq, bk,
           gamma, beta):
    xpad = jnp.pad(x, ((0, NPAD - N), (0, 0)))
    src = edge_index[0].astype(jnp.int32)
    dst = edge_index[1].astype(jnp.int32)
    loop = jnp.arange(N, dtype=jnp.int32)
    pad = jnp.full((NE_PAD - NE,), N, jnp.int32)
    dummy = jnp.full((NW, NBLK_ALLOC - NBLK, B), N, jnp.int32)
    src3 = jnp.concatenate(
        [jnp.concatenate([src, loop, pad]).reshape(NW, NBLK, B), dummy], axis=1)
    dst3 = jnp.concatenate(
        [jnp.concatenate([dst, loop, pad]).reshape(NW, NBLK, B), dummy], axis=1)

    h, s16, d16 = _prep_call(xpad, W, att_src.reshape(1, D),
                             att_dst.reshape(1, D))
    acc2, den2 = _edge_call(src3, dst3, h, s16, d16)

    wqt = Wq.transpose(1, 0, 2).reshape(N, D)
    wkt = Wk.transpose(1, 0, 2).reshape(N, D)
    hp, q_acc, k_acc = _b1_call(acc2, acc2, den2, den2, b_gat.reshape(1, D),
                                Wp, wqt, wkt)
    a_sum = _b2_call(q_acc, k_acc, bq.reshape(1, D), bk.reshape(1, D))
    return _b3_call(hp, a_sum, gamma.reshape(1, D), beta.reshape(1, D))
